# baseline (speedup 1.0000x reference)
import functools

import jax
import jax.numpy as jnp
from jax import lax
from jax.experimental import pallas as pl
from jax.experimental.pallas import tpu as pltpu
from jax.experimental.pallas import tpu_sc as plsc

_B, _N_PER_B, _F, _H = 10, 1000, 128, 128
_N = _B * _N_PER_B
_E = 320000
_NC = 2
_NS = 16
_NW = _NC * _NS
_EPW = _E // _NW
_CH = 80
_CPW = _EPW // _CH
_PCH = 128
_PCPW = 80
_SPW = _PCH * _PCPW
_PAD = _SPW - _EPW
_NH = _N + 8
_NA = _N
_WT = 10
_WR = _N // _WT
_DW = 128


def _zero_init(s, zeros_hbm, acc_sh):
    @pl.when(s < _WT)
    def _():
        pltpu.sync_copy(zeros_hbm.at[pl.ds(s * _WR, _WR)],
                        acc_sh.at[pl.ds(s * _WR, _WR)])


def _writeout(c, s, acc_sh, out_hbm):
    @pl.when(s < _WT)
    def _():
        pltpu.sync_copy(acc_sh.at[pl.ds(s * _WR, _WR)],
                        out_hbm.at[c, pl.ds(s * _WR, _WR)])


def _deg_body(dst_hbm, zeros_hbm, ones_hbm, out_hbm, dst_vv, ones_v, acc_sh):
    c = lax.axis_index("c")
    s = lax.axis_index("s")
    wid = c * _NS + s
    pltpu.sync_copy(ones_hbm, ones_v)
    pltpu.sync_copy(dst_hbm.at[wid], dst_vv)
    _zero_init(s, zeros_hbm, acc_sh)
    plsc.subcore_barrier()

    def chunk(i, carry):
        pltpu.sync_copy(ones_v, acc_sh.at[dst_vv.at[i]], add=True)
        return carry

    lax.fori_loop(0, _CPW, chunk, 0)
    plsc.subcore_barrier()
    _writeout(c, s, acc_sh, out_hbm)


def _prop_body(h_hbm, src_hbm, dst_hbm, zeros_hbm, out_hbm,
               src_v, dst_v, rows, acc_sh, ssem, dsem, gsem):
    c = lax.axis_index("c")
    s = lax.axis_index("s")
    wid = c * _NS + s
    base = wid * _SPW

    def src_copy(i, b):
        return pltpu.make_async_copy(
            src_hbm.at[pl.ds(base + i * _PCH, _PCH)], src_v[b], ssem[b])

    def dst_copy(i, b):
        return pltpu.make_async_copy(
            dst_hbm.at[pl.ds(base + i * _PCH, _PCH)], dst_v[b], dsem[b])

    def gather(b):
        return pltpu.make_async_copy(h_hbm.at[src_v[b]], rows[b], gsem[b])

    for b in range(2):
        src_copy(b, b).start()
        dst_copy(b, b).start()
    _zero_init(s, zeros_hbm, acc_sh)
    plsc.subcore_barrier()
    src_copy(0, 0).wait()
    gather(0).start()

    def step(j, carry):
        for b in range(2):
            i = j * 2 + b
            nb = 1 - b
            @pl.when(i + 1 < _PCPW)
            def _():
                src_copy(i + 1, nb).wait()
                gather(nb).start()
            gather(b).wait()
            dst_copy(i, b).wait()
            pltpu.sync_copy(rows[b], acc_sh.at[dst_v[b]], add=True)
            @pl.when(i + 2 < _PCPW)
            def _():
                src_copy(i + 2, b).start()
                dst_copy(i + 2, b).start()
        return carry

    lax.fori_loop(0, _PCPW // 2, step, 0)
    plsc.subcore_barrier()
    _writeout(c, s, acc_sh, out_hbm)


@functools.cache
def _sc_kernels():
    mesh = plsc.VectorSubcoreMesh(
        core_axis_name="c", subcore_axis_name="s",
        num_cores=_NC, num_subcores=_NS)
    deg = pl.kernel(
        _deg_body,
        out_type=jax.ShapeDtypeStruct((_NC, _N, _DW), jnp.float32),
        mesh=mesh,
        scratch_types=[
            pltpu.VMEM((_CPW, _CH), jnp.int32),
            pltpu.VMEM((_CH, _DW), jnp.float32),
            pltpu.VMEM_SHARED((_NA, _DW), jnp.float32),
        ],
    )
    prop = pl.kernel(
        _prop_body,
        out_type=jax.ShapeDtypeStruct((_NC, _N, _H), jnp.float32),
        mesh=mesh,
        scratch_types=[
            [pltpu.VMEM((_PCH,), jnp.int32)] * 2,
            [pltpu.VMEM((_PCH,), jnp.int32)] * 2,
            [pltpu.VMEM((_PCH, _H), jnp.float32)] * 2,
            pltpu.VMEM_SHARED((_NA, _H), jnp.float32),
            [pltpu.SemaphoreType.DMA] * 2,
            [pltpu.SemaphoreType.DMA] * 2,
            [pltpu.SemaphoreType.DMA] * 2,
        ],
    )
    return deg, prop


def _pre_body(x_ref, w_ref, deg_ref, h_ref, dinv_ref):
    deg = deg_ref[0, :, 0:1] + deg_ref[1, :, 0:1] + 1.0
    dinv = lax.rsqrt(deg)
    dinv_ref[...] = dinv
    h = jnp.dot(x_ref[...], w_ref[...], preferred_element_type=jnp.float32)
    h_ref[0:_N, :] = dinv * h
    h_ref[_N:_NH, :] = jnp.zeros((_NH - _N, _H), jnp.float32)


def _pre_call(x2, w0, degp):
    return pl.pallas_call(
        _pre_body,
        out_shape=(jax.ShapeDtypeStruct((_NH, _H), jnp.float32),
                   jax.ShapeDtypeStruct((_N, 1), jnp.float32)),
    )(x2, w0, degp)


def _block_body(p_ref, hprev_ref, dinv_ref, b_ref, g_ref, be_ref, w_ref,
                out_ref):
    dinv = dinv_ref[...]
    y = dinv * (p_ref[0] + p_ref[1] + hprev_ref[0:_N, :]) + b_ref[...]
    m = jnp.mean(y, axis=0, keepdims=True)
    v = jnp.mean((y - m) * (y - m), axis=0, keepdims=True)
    yn = (y - m) * lax.rsqrt(v + 1e-5) * g_ref[...] + be_ref[...]
    r = jnp.maximum(yn, 0.0)
    h = jnp.dot(r, w_ref[...], preferred_element_type=jnp.float32)
    out_ref[0:_N, :] = dinv * h
    out_ref[_N:_NH, :] = jnp.zeros((_NH - _N, _H), jnp.float32)


def _block_call(p, hprev, dinv, b, g, be, w_next):
    return pl.pallas_call(
        _block_body,
        out_shape=jax.ShapeDtypeStruct((_NH, _H), jnp.float32),
    )(p, hprev, dinv, b.reshape(1, _H), g.reshape(1, _H), be.reshape(1, _H),
      w_next)


def _final_body(p_ref, h3_ref, dinv_ref, wc_ref, bc_ref, wcls_ref, bcls_ref,
                wr_ref, br_ref, wf_ref, bf_ref, wco_ref, bco_ref,
                logits_ref, corr_ref):
    y = dinv_ref[...] * (p_ref[0] + p_ref[1] + h3_ref[0:_N, :])
    pools = [jnp.mean(y[i * _N_PER_B:(i + 1) * _N_PER_B], axis=0,
                      keepdims=True) for i in range(_B)]
    ym = jnp.concatenate(pools, axis=0)
    pooled = jnp.dot(ym, wc_ref[...],
                     preferred_element_type=jnp.float32) + bc_ref[...]
    logits_ref[...] = jnp.dot(pooled, wcls_ref[...],
                              preferred_element_type=jnp.float32) + bcls_ref[...]
    cb = jnp.dot(y, wr_ref[...],
                 preferred_element_type=jnp.float32) + br_ref[...]
    cf = jnp.maximum(
        jnp.dot(cb, wf_ref[...],
                preferred_element_type=jnp.float32) + bf_ref[...], 0.0)
    corr_ref[...] = jnp.dot(cf, wco_ref[...],
                            preferred_element_type=jnp.float32) + bco_ref[...]


def _final_call(p, h3, dinv, prm):
    return pl.pallas_call(
        _final_body,
        out_shape=(jax.ShapeDtypeStruct((_B, 12), jnp.float32),
                   jax.ShapeDtypeStruct((_N, 3), jnp.float32)),
    )(p, h3, dinv,
      prm['Wc'], prm['bc'].reshape(1, _H),
      prm['Wcls'], prm['bcls'].reshape(1, 12),
      prm['Wr'], prm['br'].reshape(1, _H),
      prm['Wf'][:_H], prm['bf'].reshape(1, _H),
      prm['Wco'], prm['bco'].reshape(1, 3))


def kernel(x, edge_index, params):
    ei = edge_index.astype(jnp.int32)
    src, dst = ei[0], ei[1]
    dst_w = dst.reshape(_NW, _CPW, _CH)
    src_p = jnp.concatenate(
        [src.reshape(_NW, _EPW),
         jnp.full((_NW, _PAD), _N, jnp.int32)], axis=1).reshape(-1)
    pad_dst = (jnp.arange(_NW * _PAD, dtype=jnp.int32) % _N).reshape(
        _NW, _PAD)
    dst_p = jnp.concatenate(
        [dst.reshape(_NW, _EPW), pad_dst], axis=1).reshape(-1)
    x2 = x.reshape(_N, _F)
    zeros_h = jnp.zeros((_N, _H), jnp.float32)
    ones_d = jnp.ones((_CH, _DW), jnp.float32)
    eye = jnp.eye(_H, dtype=jnp.float32)

    deg_kernel, prop_kernel = _sc_kernels()
    degp = deg_kernel(dst_w, zeros_h, ones_d)
    h, dinv = _pre_call(x2, params['W0'], degp)
    for i in range(3):
        p = prop_kernel(h, src_p, dst_p, zeros_h)
        w_next = params[f'W{i + 1}'] if i < 2 else eye
        h = _block_call(p, h, dinv, params[f'b{i}'], params[f'g{i}'],
                        params[f'be{i}'], w_next)
    p = prop_kernel(h, src_p, dst_p, zeros_h)
    logits, corr = _final_call(p, h, dinv, params)
    return (logits, corr.reshape(_B, _N_PER_B, 3))


# speedup vs baseline: 2.7908x; 2.7908x over previous
import functools

import jax
import jax.numpy as jnp
from jax import lax
from jax.experimental import pallas as pl
from jax.experimental.pallas import tpu as pltpu
from jax.experimental.pallas import tpu_sc as plsc

_B, _N_PER_B, _F, _H = 10, 1000, 128, 128
_N = _B * _N_PER_B
_E = 320000
_NC = 2
_NS = 16
_NW = _NC * _NS
_EPW = _E // _NW
_CH = 80
_CPW = _EPW // _CH
_NA = _N
_WT = 10
_WR = _N // _WT
_DW = 128


def _zero_init(s, zeros_hbm, acc_sh):
    @pl.when(s < _WT)
    def _():
        pltpu.sync_copy(zeros_hbm.at[pl.ds(s * _WR, _WR)],
                        acc_sh.at[pl.ds(s * _WR, _WR)])


def _writeout(c, s, acc_sh, out_hbm):
    @pl.when(s < _WT)
    def _():
        pltpu.sync_copy(acc_sh.at[pl.ds(s * _WR, _WR)],
                        out_hbm.at[c, pl.ds(s * _WR, _WR)])


def _deg_body(dst_hbm, zeros_hbm, ones_hbm, out_hbm, dst_vv, ones_v, acc_sh):
    c = lax.axis_index("c")
    s = lax.axis_index("s")
    wid = c * _NS + s
    pltpu.sync_copy(ones_hbm, ones_v)
    pltpu.sync_copy(dst_hbm.at[wid], dst_vv)
    _zero_init(s, zeros_hbm, acc_sh)
    plsc.subcore_barrier()

    def chunk(i, carry):
        pltpu.sync_copy(ones_v, acc_sh.at[dst_vv.at[i]], add=True)
        return carry

    lax.fori_loop(0, _CPW, chunk, 0)
    plsc.subcore_barrier()
    _writeout(c, s, acc_sh, out_hbm)


def _prop_body(h_hbm, src_hbm, dst_hbm, zeros_hbm, out_hbm,
               src_v, dst_vv, rows, acc_sh, gsem):
    c = lax.axis_index("c")
    s = lax.axis_index("s")
    wid = c * _NS + s
    base = wid * _EPW
    pltpu.sync_copy(src_hbm.at[pl.ds(base, _EPW)], src_v)
    pltpu.sync_copy(dst_hbm.at[wid], dst_vv)
    _zero_init(s, zeros_hbm, acc_sh)

    def gather(i, b):
        return pltpu.make_async_copy(
            h_hbm.at[src_v.at[pl.ds(i * _CH, _CH)]], rows[b], gsem[b])

    gather(0, 0).start()
    plsc.subcore_barrier()

    def step(j, carry):
        for b in range(2):
            i = j * 2 + b
            @pl.when(i + 1 < _CPW)
            def _():
                gather(i + 1, 1 - b).start()
            gather(i, b).wait()
            pltpu.sync_copy(rows[b], acc_sh.at[dst_vv.at[i]], add=True)
        return carry

    lax.fori_loop(0, _CPW // 2, step, 0)
    gather(_CPW - 1, 0).wait()
    pltpu.sync_copy(rows[0], acc_sh.at[dst_vv.at[_CPW - 1]], add=True)
    plsc.subcore_barrier()
    _writeout(c, s, acc_sh, out_hbm)


@functools.cache
def _sc_kernels():
    mesh = plsc.VectorSubcoreMesh(
        core_axis_name="c", subcore_axis_name="s",
        num_cores=_NC, num_subcores=_NS)
    deg = pl.kernel(
        _deg_body,
        out_type=jax.ShapeDtypeStruct((_NC, _N, _DW), jnp.float32),
        mesh=mesh,
        scratch_types=[
            pltpu.VMEM((_CPW, _CH), jnp.int32),
            pltpu.VMEM((_CH, _DW), jnp.float32),
            pltpu.VMEM_SHARED((_NA, _DW), jnp.float32),
        ],
    )
    prop = pl.kernel(
        _prop_body,
        out_type=jax.ShapeDtypeStruct((_NC, _N, _H), jnp.float32),
        mesh=mesh,
        scratch_types=[
            pltpu.VMEM((_EPW,), jnp.int32),
            pltpu.VMEM((_CPW, _CH), jnp.int32),
            [pltpu.VMEM((_CH, _H), jnp.float32)] * 2,
            pltpu.VMEM_SHARED((_NA, _H), jnp.float32),
            [pltpu.SemaphoreType.DMA] * 2,
        ],
    )
    return deg, prop


def _pre_body(x_ref, w_ref, deg_ref, h_ref, dinv_ref):
    deg = deg_ref[0, :, 0:1] + deg_ref[1, :, 0:1] + 1.0
    dinv = lax.rsqrt(deg)
    dinv_ref[...] = dinv
    h = jnp.dot(x_ref[...], w_ref[...], preferred_element_type=jnp.float32)
    h_ref[...] = dinv * h


def _pre_call(x2, w0, degp):
    return pl.pallas_call(
        _pre_body,
        out_shape=(jax.ShapeDtypeStruct((_N, _H), jnp.float32),
                   jax.ShapeDtypeStruct((_N, 1), jnp.float32)),
    )(x2, w0, degp)


def _block_body(p_ref, hprev_ref, dinv_ref, b_ref, g_ref, be_ref, w_ref,
                out_ref):
    dinv = dinv_ref[...]
    y = dinv * (p_ref[0] + p_ref[1] + hprev_ref[...]) + b_ref[...]
    m = jnp.mean(y, axis=0, keepdims=True)
    v = jnp.mean((y - m) * (y - m), axis=0, keepdims=True)
    yn = (y - m) * lax.rsqrt(v + 1e-5) * g_ref[...] + be_ref[...]
    r = jnp.maximum(yn, 0.0)
    h = jnp.dot(r, w_ref[...], preferred_element_type=jnp.float32)
    out_ref[...] = dinv * h


def _block_call(p, hprev, dinv, b, g, be, w_next):
    return pl.pallas_call(
        _block_body,
        out_shape=jax.ShapeDtypeStruct((_N, _H), jnp.float32),
    )(p, hprev, dinv, b.reshape(1, _H), g.reshape(1, _H), be.reshape(1, _H),
      w_next)


def _final_body(p_ref, h3_ref, dinv_ref, wc_ref, bc_ref, wcls_ref, bcls_ref,
                wr_ref, br_ref, wf_ref, bf_ref, wco_ref, bco_ref,
                logits_ref, corr_ref):
    y = dinv_ref[...] * (p_ref[0] + p_ref[1] + h3_ref[...])
    pools = [jnp.mean(y[i * _N_PER_B:(i + 1) * _N_PER_B], axis=0,
                      keepdims=True) for i in range(_B)]
    ym = jnp.concatenate(pools, axis=0)
    pooled = jnp.dot(ym, wc_ref[...],
                     preferred_element_type=jnp.float32) + bc_ref[...]
    logits_ref[...] = jnp.dot(pooled, wcls_ref[...],
                              preferred_element_type=jnp.float32) + bcls_ref[...]
    cb = jnp.dot(y, wr_ref[...],
                 preferred_element_type=jnp.float32) + br_ref[...]
    cf = jnp.maximum(
        jnp.dot(cb, wf_ref[...],
                preferred_element_type=jnp.float32) + bf_ref[...], 0.0)
    corr_ref[...] = jnp.dot(cf, wco_ref[...],
                            preferred_element_type=jnp.float32) + bco_ref[...]


def _final_call(p, h3, dinv, prm):
    return pl.pallas_call(
        _final_body,
        out_shape=(jax.ShapeDtypeStruct((_B, 12), jnp.float32),
                   jax.ShapeDtypeStruct((_N, 3), jnp.float32)),
    )(p, h3, dinv,
      prm['Wc'], prm['bc'].reshape(1, _H),
      prm['Wcls'], prm['bcls'].reshape(1, 12),
      prm['Wr'], prm['br'].reshape(1, _H),
      prm['Wf'][:_H], prm['bf'].reshape(1, _H),
      prm['Wco'], prm['bco'].reshape(1, 3))


def kernel(x, edge_index, params):
    ei = edge_index.astype(jnp.int32)
    src, dst = ei[0], ei[1]
    dst_w = dst.reshape(_NW, _CPW, _CH)
    x2 = x.reshape(_N, _F)
    zeros_h = jnp.zeros((_N, _H), jnp.float32)
    ones_d = jnp.ones((_CH, _DW), jnp.float32)
    eye = jnp.eye(_H, dtype=jnp.float32)

    deg_kernel, prop_kernel = _sc_kernels()
    degp = deg_kernel(dst_w, zeros_h, ones_d)
    h, dinv = _pre_call(x2, params['W0'], degp)
    for i in range(3):
        p = prop_kernel(h, src, dst_w, zeros_h)
        w_next = params[f'W{i + 1}'] if i < 2 else eye
        h = _block_call(p, h, dinv, params[f'b{i}'], params[f'g{i}'],
                        params[f'be{i}'], w_next)
    p = prop_kernel(h, src, dst_w, zeros_h)
    logits, corr = _final_call(p, h, dinv, params)
    return (logits, corr.reshape(_B, _N_PER_B, 3))
